# Initial kernel scaffold; baseline (speedup 1.0000x reference)
import jax
import jax.numpy as jnp
from jax.experimental import pallas as pl


def kernel(x, tok_table, pos_table):
    raise NotImplementedError("write your pallas kernel here")


# speedup vs baseline: 1.2717x; 1.2717x over previous
import functools

import jax
import jax.numpy as jnp
from jax import lax
from jax.experimental import pallas as pl
from jax.experimental.pallas import tpu as pltpu
from jax.experimental.pallas import tpu_sc as plsc

_NUM_CORES = 2
_NUM_SUBCORES = 16
_NW = _NUM_CORES * _NUM_SUBCORES
_LANES = 16
_GATHER_CHUNK = 128


@functools.lru_cache(maxsize=None)
def _build(n_rows, seq_len, dim):
  rows_per_w = n_rows // _NW
  n_chunks = rows_per_w // _GATHER_CHUNK
  vecs_per_row = dim // _LANES

  mesh = plsc.VectorSubcoreMesh(core_axis_name="c", subcore_axis_name="s")

  @functools.partial(
      pl.kernel,
      mesh=mesh,
      out_type=jax.ShapeDtypeStruct((n_rows, dim), jnp.float32),
      scratch_types=[
          pltpu.VMEM((rows_per_w,), jnp.int32),
          pltpu.VMEM((rows_per_w, dim), jnp.float32),
          pltpu.VMEM((rows_per_w, dim), jnp.float32),
          pltpu.SemaphoreType.DMA,
          pltpu.SemaphoreType.DMA,
      ],
  )
  def tok_pos_embed(idx_hbm, tok_hbm, pos_hbm, out_hbm,
                    idx_v, rows_v, pos_v, gsem, psem):
    wid = lax.axis_index("s") * _NUM_CORES + lax.axis_index("c")
    base = wid * rows_per_w
    s_base = lax.rem(base, seq_len)

    pltpu.sync_copy(idx_hbm.at[pl.ds(base, rows_per_w)], idx_v)
    pos_cp = pltpu.async_copy(pos_hbm.at[pl.ds(s_base, rows_per_w)], pos_v, psem)
    gathers = []
    for j in range(n_chunks):
      sl = pl.ds(j * _GATHER_CHUNK, _GATHER_CHUNK)
      gathers.append(
          pltpu.async_copy(tok_hbm.at[idx_v.at[sl]], rows_v.at[sl], gsem))
    pos_cp.wait()
    for g in gathers:
      g.wait()

    def add_row(r, carry):
      for c in range(vecs_per_row):
        sl = pl.ds(c * _LANES, _LANES)
        rows_v[r, sl] = rows_v[r, sl] + pos_v[r, sl]
      return carry

    lax.fori_loop(0, rows_per_w, add_row, 0)

    pltpu.sync_copy(rows_v, out_hbm.at[pl.ds(base, rows_per_w)])

  return tok_pos_embed


def kernel(x, tok_table, pos_table):
  batch, seq_len = x.shape
  _, dim = tok_table.shape
  n_rows = batch * seq_len
  fn = _build(n_rows, seq_len, dim)
  flat = fn(x.reshape(n_rows).astype(jnp.int32), tok_table, pos_table)
  return flat.reshape(batch, seq_len, dim)


# speedup vs baseline: 1.3673x; 1.0752x over previous
import functools

import jax
import jax.numpy as jnp
from jax import lax
from jax.experimental import pallas as pl
from jax.experimental.pallas import tpu as pltpu
from jax.experimental.pallas import tpu_sc as plsc

_NUM_CORES = 2
_NUM_SUBCORES = 16
_NW = _NUM_CORES * _NUM_SUBCORES
_LANES = 16
_GATHER_CHUNK = 128


@functools.lru_cache(maxsize=None)
def _build(n_rows, seq_len, dim):
  rows_per_w = n_rows // _NW
  n_chunks = rows_per_w // _GATHER_CHUNK
  vecs_per_row = dim // _LANES

  mesh = plsc.VectorSubcoreMesh(core_axis_name="c", subcore_axis_name="s")

  @functools.partial(
      pl.kernel,
      mesh=mesh,
      out_type=jax.ShapeDtypeStruct((n_rows, dim), jnp.float32),
      scratch_types=[
          pltpu.VMEM((rows_per_w,), jnp.int32),
          pltpu.VMEM((rows_per_w, dim), jnp.float32),
          pltpu.SemaphoreType.DMA,
          pltpu.SemaphoreType.DMA,
      ],
  )
  def tok_pos_embed(idx_hbm, tok_hbm, pos_hbm, out_hbm,
                    idx_v, rows_v, gsem, psem):
    wid = lax.axis_index("s") * _NUM_CORES + lax.axis_index("c")
    base = wid * rows_per_w
    s_base = lax.rem(base, seq_len)

    idx_cp = pltpu.async_copy(idx_hbm.at[pl.ds(base, rows_per_w)], idx_v, psem)
    pltpu.sync_copy(pos_hbm.at[pl.ds(s_base, rows_per_w)], rows_v)
    idx_cp.wait()
    gathers = []
    for j in range(n_chunks):
      sl = pl.ds(j * _GATHER_CHUNK, _GATHER_CHUNK)
      gathers.append(
          pltpu.async_copy(tok_hbm.at[idx_v.at[sl]], rows_v.at[sl], gsem,
                           add=True))
    for g in gathers:
      g.wait()

    pltpu.sync_copy(rows_v, out_hbm.at[pl.ds(base, rows_per_w)])

  return tok_pos_embed


def kernel(x, tok_table, pos_table):
  batch, seq_len = x.shape
  _, dim = tok_table.shape
  n_rows = batch * seq_len
  fn = _build(n_rows, seq_len, dim)
  flat = fn(x.reshape(n_rows).astype(jnp.int32), tok_table, pos_table)
  return flat.reshape(batch, seq_len, dim)


# speedup vs baseline: 1.3719x; 1.0034x over previous
import functools

import jax
import jax.numpy as jnp
from jax import lax
from jax.experimental import pallas as pl
from jax.experimental.pallas import tpu as pltpu
from jax.experimental.pallas import tpu_sc as plsc

_NUM_CORES = 2
_NUM_SUBCORES = 16
_NW = _NUM_CORES * _NUM_SUBCORES
_GATHER_CHUNK = 128


@functools.lru_cache(maxsize=None)
def _build(batch, seq_len, dim):
  n_rows = batch * seq_len
  rows_per_w = n_rows // _NW
  n_chunks = rows_per_w // _GATHER_CHUNK

  mesh = plsc.VectorSubcoreMesh(core_axis_name="c", subcore_axis_name="s")

  @functools.partial(
      pl.kernel,
      mesh=mesh,
      out_type=jax.ShapeDtypeStruct((batch, seq_len, dim), jnp.float32),
      scratch_types=[
          pltpu.VMEM((rows_per_w,), jnp.int32),
          pltpu.VMEM((rows_per_w, dim), jnp.float32),
          pltpu.SemaphoreType.DMA,
          pltpu.SemaphoreType.DMA,
      ],
  )
  def tok_pos_embed(idx_hbm, tok_hbm, pos_hbm, out_hbm,
                    idx_v, rows_v, gsem, psem):
    wid = lax.axis_index("s") * _NUM_CORES + lax.axis_index("c")
    base = wid * rows_per_w
    b = lax.div(base, seq_len)
    s_base = lax.rem(base, seq_len)

    idx_cp = pltpu.async_copy(idx_hbm.at[b, pl.ds(s_base, rows_per_w)],
                              idx_v, psem)
    pltpu.sync_copy(pos_hbm.at[pl.ds(s_base, rows_per_w)], rows_v)
    idx_cp.wait()
    gathers = []
    for j in range(n_chunks):
      sl = pl.ds(j * _GATHER_CHUNK, _GATHER_CHUNK)
      gathers.append(
          pltpu.async_copy(tok_hbm.at[idx_v.at[sl]], rows_v.at[sl], gsem,
                           add=True))
    for g in gathers:
      g.wait()

    pltpu.sync_copy(rows_v, out_hbm.at[b, pl.ds(s_base, rows_per_w)])

  return tok_pos_embed


def kernel(x, tok_table, pos_table):
  batch, seq_len = x.shape
  _, dim = tok_table.shape
  fn = _build(batch, seq_len, dim)
  return fn(x, tok_table, pos_table)


# speedup vs baseline: 1.3779x; 1.0044x over previous
import functools

import jax
import jax.numpy as jnp
from jax import lax
from jax.experimental import pallas as pl
from jax.experimental.pallas import tpu as pltpu
from jax.experimental.pallas import tpu_sc as plsc

_NUM_CORES = 2
_NUM_SUBCORES = 16
_NW = _NUM_CORES * _NUM_SUBCORES
_GATHER_CHUNK = 128


@functools.lru_cache(maxsize=None)
def _build(batch, seq_len, dim):
  n_rows = batch * seq_len
  rows_per_w = n_rows // _NW
  n_chunks = rows_per_w // _GATHER_CHUNK

  mesh = plsc.VectorSubcoreMesh(core_axis_name="c", subcore_axis_name="s")

  @functools.partial(
      pl.kernel,
      mesh=mesh,
      out_type=jax.ShapeDtypeStruct((batch, seq_len, dim), jnp.float32),
      scratch_types=(
          [pltpu.VMEM((rows_per_w,), jnp.int32),
           pltpu.VMEM((rows_per_w, dim), jnp.float32)]
          + [pltpu.SemaphoreType.DMA] * (2 * n_chunks + 2)
      ),
  )
  def tok_pos_embed(idx_hbm, tok_hbm, pos_hbm, out_hbm,
                    idx_v, rows_v, *sems):
    isem = sems[0]
    psems = sems[1:1 + n_chunks]
    gsems = sems[1 + n_chunks:1 + 2 * n_chunks]
    osem = sems[1 + 2 * n_chunks]

    wid = lax.axis_index("s") * _NUM_CORES + lax.axis_index("c")
    base = wid * rows_per_w
    b = lax.div(base, seq_len)
    s_base = lax.rem(base, seq_len)

    idx_cp = pltpu.async_copy(idx_hbm.at[b, pl.ds(s_base, rows_per_w)],
                              idx_v, isem)
    pos_cps = []
    for j in range(n_chunks):
      sl = pl.ds(j * _GATHER_CHUNK, _GATHER_CHUNK)
      pos_cps.append(
          pltpu.async_copy(pos_hbm.at[pl.ds(s_base + j * _GATHER_CHUNK,
                                            _GATHER_CHUNK)],
                           rows_v.at[sl], psems[j]))
    idx_cp.wait()
    gathers = []
    for j in range(n_chunks):
      sl = pl.ds(j * _GATHER_CHUNK, _GATHER_CHUNK)
      pos_cps[j].wait()
      gathers.append(
          pltpu.async_copy(tok_hbm.at[idx_v.at[sl]], rows_v.at[sl],
                           gsems[j], add=True))
    stores = []
    for j in range(n_chunks):
      sl = pl.ds(j * _GATHER_CHUNK, _GATHER_CHUNK)
      gathers[j].wait()
      stores.append(
          pltpu.async_copy(rows_v.at[sl],
                           out_hbm.at[b, pl.ds(s_base + j * _GATHER_CHUNK,
                                               _GATHER_CHUNK)],
                           osem))
    for s in stores:
      s.wait()

  return tok_pos_embed


def kernel(x, tok_table, pos_table):
  batch, seq_len = x.shape
  _, dim = tok_table.shape
  fn = _build(batch, seq_len, dim)
  return fn(x, tok_table, pos_table)
